# baseline (speedup 1.0000x reference)
import functools

import jax
import jax.numpy as jnp
from jax import lax
from jax.experimental import pallas as pl
from jax.experimental.pallas import tpu as pltpu
from jax.experimental.pallas import tpu_sc as plsc

WIN = 1024
HOP = 256
NFFT = 1024
FBINS = 513
T = 4097
FP = 640
TB = 272
GRID = 16
TP = TB * GRID
CROWS = 4368
NW = 32
CHUNK = TP // NW
FB = 8
NB = CHUNK // FB
BATCH = FB * FP
NVEC = BATCH // 16


def _frames_body(c_ref, h_ref, fr_ref):
    i = pl.program_id(0)
    s = i * TB
    rows = c_ref[pl.ds(s, TB + 8), :]
    fr = jnp.concatenate(
        [rows[0:TB], rows[1:TB + 1], rows[2:TB + 2], rows[3:TB + 3]], axis=1)
    fr_ref[...] = fr * h_ref[0:1, :]


def _frames_stage(c, h2, interpret=False):
    return pl.pallas_call(
        _frames_body,
        grid=(GRID,),
        in_specs=[
            pl.BlockSpec((CROWS, HOP), lambda i: (0, 0)),
            pl.BlockSpec((8, NFFT), lambda i: (0, 0)),
        ],
        out_specs=pl.BlockSpec((TB, NFFT), lambda i: (i, 0)),
        out_shape=jax.ShapeDtypeStruct((TP, NFFT), jnp.float32),
        interpret=interpret,
    )(c, h2)


def _reassign_body(re_ref, im_ref, inst_ref, vre_ref, vim_ref, kk_ref):
    re = re_ref[...]
    im = im_ref[...]
    inst = inst_ref[...]
    fint = lax.broadcasted_iota(jnp.int32, (TB, FP), 1)
    fcol = fint.astype(jnp.float32)
    k = (fcol + inst).astype(jnp.int32)
    real_f = fint < FBINS
    valid = (k >= 0) & (k < FBINS) & real_f
    kc = jnp.where(real_f, jnp.clip(k, 0, FBINS - 1), fint)
    rr = lax.broadcasted_iota(jnp.int32, (TB, FP), 0)
    vre_ref[...] = jnp.where(valid, re, 0.0)
    vim_ref[...] = jnp.where(valid, im, 0.0)
    kk_ref[...] = kc + (rr & (FB - 1)) * FP


def _reassign_stage(re, im, inst, interpret=False):
    spec = pl.BlockSpec((TB, FP), lambda i: (i, 0))
    return pl.pallas_call(
        _reassign_body,
        grid=(GRID,),
        in_specs=[spec, spec, spec],
        out_specs=[spec, spec, spec],
        out_shape=[
            jax.ShapeDtypeStruct((TP, FP), jnp.float32),
            jax.ShapeDtypeStruct((TP, FP), jnp.float32),
            jax.ShapeDtypeStruct((TP, FP), jnp.int32),
        ],
        interpret=interpret,
    )(re, im, inst)


def _sc_scatter_body(vre_hbm, vim_hbm, kk_hbm, outre_hbm, outim_hbm,
                     bre, bim, bk, are, aim, sem):
    wid = lax.axis_index("s") * 2 + lax.axis_index("c")
    base = wid * CHUNK * FP
    zero = jnp.zeros((16,), jnp.float32)

    def batch(it, carry):
        off = base + it * BATCH
        pltpu.sync_copy(vre_hbm.at[pl.ds(off, BATCH)], bre)
        pltpu.sync_copy(vim_hbm.at[pl.ds(off, BATCH)], bim)
        pltpu.sync_copy(kk_hbm.at[pl.ds(off, BATCH)], bk)
        for j in range(NVEC):
            sl = pl.ds(j * 16, 16)
            are[sl] = zero
            aim[sl] = zero
        for j in range(NVEC):
            sl = pl.ds(j * 16, 16)
            idx = bk[sl]
            plsc.addupdate_scatter(are, [idx], bre[sl])
            plsc.addupdate_scatter(aim, [idx], bim[sl])
        pltpu.sync_copy(are, outre_hbm.at[pl.ds(off, BATCH)])
        pltpu.sync_copy(aim, outim_hbm.at[pl.ds(off, BATCH)])
        return carry

    lax.fori_loop(0, NB, batch, 0)


@functools.cache
def _sc_scatter():
    mesh = plsc.VectorSubcoreMesh(core_axis_name="c", subcore_axis_name="s")
    return pl.kernel(
        _sc_scatter_body,
        out_type=[
            jax.ShapeDtypeStruct((TP * FP,), jnp.float32),
            jax.ShapeDtypeStruct((TP * FP,), jnp.float32),
        ],
        mesh=mesh,
        scratch_types=[
            pltpu.VMEM((BATCH,), jnp.float32),
            pltpu.VMEM((BATCH,), jnp.float32),
            pltpu.VMEM((BATCH,), jnp.int32),
            pltpu.VMEM((BATCH,), jnp.float32),
            pltpu.VMEM((BATCH,), jnp.float32),
            pltpu.SemaphoreType.DMA,
        ],
        compiler_params=pltpu.CompilerParams(
            use_tc_tiling_on_sc=False, needs_layout_passes=False),
    )


def _pad2(a):
    return jnp.pad(a, ((0, TP - T), (0, FP - FBINS)))


def kernel(x):
    n = jnp.arange(WIN, dtype=jnp.float32)
    hann = 0.5 - 0.5 * jnp.cos(2.0 * jnp.pi * n / WIN)
    h2 = jnp.broadcast_to(hann[None, :], (8, NFFT))
    xp = jnp.pad(x, (NFFT // 2, NFFT // 2), mode="reflect")
    c = jnp.zeros((CROWS, HOP), jnp.float32).at[:xp.shape[0] // HOP].set(
        xp.reshape(-1, HOP))
    frames = _frames_stage(c, h2)
    spec = jnp.fft.rfft(frames[:T], n=NFFT, axis=-1)
    ph = jnp.angle(spec)
    inst = jnp.diff(ph, axis=0)
    instp = jnp.concatenate([inst, inst[-1:]], axis=0)
    vre, vim, kk = _reassign_stage(
        _pad2(spec.real), _pad2(spec.imag), _pad2(instp))
    outre, outim = _sc_scatter()(
        vre.reshape(TP * FP), vim.reshape(TP * FP), kk.reshape(TP * FP))
    outre = outre.reshape(TP, FP)
    outim = outim.reshape(TP, FP)
    return lax.complex(outre[:T, :FBINS].T, outim[:T, :FBINS].T)


# speedup vs baseline: 1.0304x; 1.0304x over previous
import functools

import jax
import jax.numpy as jnp
from jax import lax
from jax.experimental import pallas as pl
from jax.experimental.pallas import tpu as pltpu
from jax.experimental.pallas import tpu_sc as plsc

WIN = 1024
HOP = 256
NFFT = 1024
FBINS = 513
T = 4097
FP = 640
TB = 288
GRID = 16
TP = TB * GRID
CROWS = 4624
NW = 32
CHUNK = TP // NW
FB = 8
NB = CHUNK // FB
BATCH = FB * FP
NVEC = BATCH // 16


def _frames_body(c_ref, h_ref, fr_ref):
    i = pl.program_id(0)
    s = i * TB
    rows = c_ref[pl.ds(s, TB + 8), :]
    fr = jnp.concatenate(
        [rows[0:TB], rows[1:TB + 1], rows[2:TB + 2], rows[3:TB + 3]], axis=1)
    fr_ref[...] = fr * h_ref[0:1, :]


def _frames_stage(c, h2, interpret=False):
    return pl.pallas_call(
        _frames_body,
        grid=(GRID,),
        in_specs=[
            pl.BlockSpec((CROWS, HOP), lambda i: (0, 0)),
            pl.BlockSpec((8, NFFT), lambda i: (0, 0)),
        ],
        out_specs=pl.BlockSpec((TB, NFFT), lambda i: (i, 0)),
        out_shape=jax.ShapeDtypeStruct((TP, NFFT), jnp.float32),
        interpret=interpret,
    )(c, h2)


def _reassign_body(re_ref, im_ref, inst_ref, vre_ref, vim_ref, kk_ref):
    re = re_ref[...]
    im = im_ref[...]
    inst = inst_ref[...]
    fint = lax.broadcasted_iota(jnp.int32, (TB, FP), 1)
    fcol = fint.astype(jnp.float32)
    k = (fcol + inst).astype(jnp.int32)
    real_f = fint < FBINS
    valid = (k >= 0) & (k < FBINS) & real_f
    kc = jnp.where(real_f, jnp.clip(k, 0, FBINS - 1), fint)
    rr = lax.broadcasted_iota(jnp.int32, (TB, FP), 0)
    vre_ref[...] = jnp.where(valid, re, 0.0)
    vim_ref[...] = jnp.where(valid, im, 0.0)
    kk_ref[...] = kc + (rr & (FB - 1)) * FP


def _reassign_stage(re, im, inst, interpret=False):
    spec = pl.BlockSpec((TB, FP), lambda i: (i, 0))
    return pl.pallas_call(
        _reassign_body,
        grid=(GRID,),
        in_specs=[spec, spec, spec],
        out_specs=[spec, spec, spec],
        out_shape=[
            jax.ShapeDtypeStruct((TP, FP), jnp.float32),
            jax.ShapeDtypeStruct((TP, FP), jnp.float32),
            jax.ShapeDtypeStruct((TP, FP), jnp.int32),
        ],
        interpret=interpret,
    )(re, im, inst)


def _sc_scatter_body(vre_hbm, vim_hbm, kk_hbm, outre_hbm, outim_hbm,
                     bre, bim, bk, are, aim, insem, outsem):
    wid = lax.axis_index("s") * 2 + lax.axis_index("c")
    base = wid * CHUNK * FP
    zero = jnp.zeros((16,), jnp.float32)

    def start_in(b, p):
        off = base + b * BATCH
        pltpu.async_copy(vre_hbm.at[pl.ds(off, BATCH)], bre[p], insem[p])
        pltpu.async_copy(vim_hbm.at[pl.ds(off, BATCH)], bim[p], insem[p])
        pltpu.async_copy(kk_hbm.at[pl.ds(off, BATCH)], bk[p], insem[p])

    def wait_in(b, p):
        off = base + b * BATCH
        pltpu.make_async_copy(vre_hbm.at[pl.ds(off, BATCH)], bre[p],
                              insem[p]).wait()
        pltpu.make_async_copy(vim_hbm.at[pl.ds(off, BATCH)], bim[p],
                              insem[p]).wait()
        pltpu.make_async_copy(kk_hbm.at[pl.ds(off, BATCH)], bk[p],
                              insem[p]).wait()

    def wait_out(b, p):
        off = base + b * BATCH
        pltpu.make_async_copy(are[p], outre_hbm.at[pl.ds(off, BATCH)],
                              outsem[p]).wait()
        pltpu.make_async_copy(aim[p], outim_hbm.at[pl.ds(off, BATCH)],
                              outsem[p]).wait()

    start_in(0, 0)

    def pair(it, carry):
        for p in (0, 1):
            b = it * 2 + p
            @pl.when(b + 1 < NB)
            def _():
                start_in(b + 1, 1 - p)
            wait_in(b, p)
            @pl.when(b >= 2)
            def _():
                wait_out(b - 2, p)
            for j in range(NVEC):
                sl = pl.ds(j * 16, 16)
                are[p][sl] = zero
                aim[p][sl] = zero
            for j in range(NVEC):
                sl = pl.ds(j * 16, 16)
                idx = bk[p][sl]
                plsc.addupdate_scatter(are[p], [idx], bre[p][sl])
                plsc.addupdate_scatter(aim[p], [idx], bim[p][sl])
            off = base + b * BATCH
            pltpu.async_copy(are[p], outre_hbm.at[pl.ds(off, BATCH)],
                             outsem[p])
            pltpu.async_copy(aim[p], outim_hbm.at[pl.ds(off, BATCH)],
                             outsem[p])
        return carry

    lax.fori_loop(0, NB // 2, pair, 0)
    wait_out(NB - 2, 0)
    wait_out(NB - 1, 1)


@functools.cache
def _sc_scatter():
    mesh = plsc.VectorSubcoreMesh(core_axis_name="c", subcore_axis_name="s")
    vmem_f = pltpu.VMEM((BATCH,), jnp.float32)
    vmem_i = pltpu.VMEM((BATCH,), jnp.int32)
    return pl.kernel(
        _sc_scatter_body,
        out_type=[
            jax.ShapeDtypeStruct((TP * FP,), jnp.float32),
            jax.ShapeDtypeStruct((TP * FP,), jnp.float32),
        ],
        mesh=mesh,
        scratch_types=[
            (vmem_f, vmem_f),
            (vmem_f, vmem_f),
            (vmem_i, vmem_i),
            (vmem_f, vmem_f),
            (vmem_f, vmem_f),
            (pltpu.SemaphoreType.DMA, pltpu.SemaphoreType.DMA),
            (pltpu.SemaphoreType.DMA, pltpu.SemaphoreType.DMA),
        ],
        compiler_params=pltpu.CompilerParams(
            use_tc_tiling_on_sc=False, needs_layout_passes=False),
    )


def _pad2(a):
    return jnp.pad(a, ((0, TP - T), (0, FP - FBINS)))


def kernel(x):
    n = jnp.arange(WIN, dtype=jnp.float32)
    hann = 0.5 - 0.5 * jnp.cos(2.0 * jnp.pi * n / WIN)
    h2 = jnp.broadcast_to(hann[None, :], (8, NFFT))
    xp = jnp.pad(x, (NFFT // 2, NFFT // 2), mode="reflect")
    c = jnp.zeros((CROWS, HOP), jnp.float32).at[:xp.shape[0] // HOP].set(
        xp.reshape(-1, HOP))
    frames = _frames_stage(c, h2)
    spec = jnp.fft.rfft(frames[:T], n=NFFT, axis=-1)
    ph = jnp.angle(spec)
    inst = jnp.diff(ph, axis=0)
    instp = jnp.concatenate([inst, inst[-1:]], axis=0)
    vre, vim, kk = _reassign_stage(
        _pad2(spec.real), _pad2(spec.imag), _pad2(instp))
    outre, outim = _sc_scatter()(
        vre.reshape(TP * FP), vim.reshape(TP * FP), kk.reshape(TP * FP))
    outre = outre.reshape(TP, FP)
    outim = outim.reshape(TP, FP)
    return lax.complex(outre[:T, :FBINS].T, outim[:T, :FBINS].T)


# speedup vs baseline: 1.0552x; 1.0241x over previous
import functools

import jax
import jax.numpy as jnp
from jax import lax
from jax.experimental import pallas as pl
from jax.experimental.pallas import tpu as pltpu
from jax.experimental.pallas import tpu_sc as plsc

WIN = 1024
HOP = 256
NFFT = 1024
FBINS = 513
T = 4097
FP = 640
TB = 288
GRID = 16
TP = TB * GRID
CROWS = 4624
NW = 32
CHUNK = TP // NW
FB = 8
NB = CHUNK // FB
BATCH = FB * FP
NVEC = BATCH // 16


def _frames_body(c_ref, h_ref, fr_ref):
    i = pl.program_id(0)
    s = i * TB
    rows = c_ref[pl.ds(s, TB + 8), :]
    fr = jnp.concatenate(
        [rows[0:TB], rows[1:TB + 1], rows[2:TB + 2], rows[3:TB + 3]], axis=1)
    fr_ref[...] = fr * h_ref[0:1, :]


def _frames_stage(c, h2, interpret=False):
    return pl.pallas_call(
        _frames_body,
        grid=(GRID,),
        in_specs=[
            pl.BlockSpec((CROWS, HOP), lambda i: (0, 0)),
            pl.BlockSpec((8, NFFT), lambda i: (0, 0)),
        ],
        out_specs=pl.BlockSpec((TB, NFFT), lambda i: (i, 0)),
        out_shape=jax.ShapeDtypeStruct((TP, NFFT), jnp.float32),
        interpret=interpret,
    )(c, h2)


def _reassign_body(re_ref, im_ref, inst_ref, vre_ref, vim_ref, kk_ref):
    re = re_ref[...]
    im = im_ref[...]
    inst = inst_ref[...]
    fint = lax.broadcasted_iota(jnp.int32, (TB, FP), 1)
    fcol = fint.astype(jnp.float32)
    k = (fcol + inst).astype(jnp.int32)
    real_f = fint < FBINS
    valid = (k >= 0) & (k < FBINS) & real_f
    kc = jnp.where(real_f, jnp.clip(k, 0, FBINS - 1), fint)
    rr = lax.broadcasted_iota(jnp.int32, (TB, FP), 0)
    vre_ref[...] = jnp.where(valid, re, 0.0)
    vim_ref[...] = jnp.where(valid, im, 0.0)
    kk_ref[...] = kc + (rr & (FB - 1)) * FP


def _reassign_stage(re, im, inst, interpret=False):
    ispec = pl.BlockSpec((TB, FP), lambda i: (i, 0))
    return pl.pallas_call(
        _reassign_body,
        grid=(GRID,),
        in_specs=[ispec, ispec, ispec],
        out_specs=[ispec, ispec, ispec],
        out_shape=[
            jax.ShapeDtypeStruct((TP, FP), jnp.float32),
            jax.ShapeDtypeStruct((TP, FP), jnp.float32),
            jax.ShapeDtypeStruct((TP, FP), jnp.int32),
        ],
        interpret=interpret,
    )(re, im, inst)


def _sc_scatter_body(vre_hbm, vim_hbm, kk_hbm, outre_hbm, outim_hbm,
                     bre, bim, bk, are, aim, insem, outsem):
    wid = lax.axis_index("s") * 2 + lax.axis_index("c")
    base = wid * CHUNK * FP
    zero = jnp.zeros((16,), jnp.float32)

    def start_in(b, p):
        off = base + b * BATCH
        pltpu.async_copy(vre_hbm.at[pl.ds(off, BATCH)], bre[p], insem[p])
        pltpu.async_copy(vim_hbm.at[pl.ds(off, BATCH)], bim[p], insem[p])
        pltpu.async_copy(kk_hbm.at[pl.ds(off, BATCH)], bk[p], insem[p])

    def wait_in(b, p):
        off = base + b * BATCH
        pltpu.make_async_copy(vre_hbm.at[pl.ds(off, BATCH)], bre[p],
                              insem[p]).wait()
        pltpu.make_async_copy(vim_hbm.at[pl.ds(off, BATCH)], bim[p],
                              insem[p]).wait()
        pltpu.make_async_copy(kk_hbm.at[pl.ds(off, BATCH)], bk[p],
                              insem[p]).wait()

    def wait_out(b, p):
        off = base + b * BATCH
        pltpu.make_async_copy(are[p], outre_hbm.at[pl.ds(off, BATCH)],
                              outsem[p]).wait()
        pltpu.make_async_copy(aim[p], outim_hbm.at[pl.ds(off, BATCH)],
                              outsem[p]).wait()

    start_in(0, 0)

    def pair(it, carry):
        for p in (0, 1):
            b = it * 2 + p
            @pl.when(b + 1 < NB)
            def _():
                start_in(b + 1, 1 - p)
            wait_in(b, p)
            @pl.when(b >= 2)
            def _():
                wait_out(b - 2, p)
            for j in range(NVEC):
                if j % (FP // 16) >= 33:
                    continue
                sl = pl.ds(j * 16, 16)
                are[p][sl] = zero
                aim[p][sl] = zero
            for j in range(NVEC):
                if j % (FP // 16) >= 33:
                    continue
                sl = pl.ds(j * 16, 16)
                idx = bk[p][sl]
                plsc.addupdate_scatter(are[p], [idx], bre[p][sl])
                plsc.addupdate_scatter(aim[p], [idx], bim[p][sl])
            off = base + b * BATCH
            pltpu.async_copy(are[p], outre_hbm.at[pl.ds(off, BATCH)],
                             outsem[p])
            pltpu.async_copy(aim[p], outim_hbm.at[pl.ds(off, BATCH)],
                             outsem[p])
        return carry

    lax.fori_loop(0, NB // 2, pair, 0)
    wait_out(NB - 2, 0)
    wait_out(NB - 1, 1)


@functools.cache
def _sc_scatter():
    mesh = plsc.VectorSubcoreMesh(core_axis_name="c", subcore_axis_name="s")
    vmem_f = pltpu.VMEM((BATCH,), jnp.float32)
    vmem_i = pltpu.VMEM((BATCH,), jnp.int32)
    return pl.kernel(
        _sc_scatter_body,
        out_type=[
            jax.ShapeDtypeStruct((TP * FP,), jnp.float32),
            jax.ShapeDtypeStruct((TP * FP,), jnp.float32),
        ],
        mesh=mesh,
        scratch_types=[
            (vmem_f, vmem_f),
            (vmem_f, vmem_f),
            (vmem_i, vmem_i),
            (vmem_f, vmem_f),
            (vmem_f, vmem_f),
            (pltpu.SemaphoreType.DMA, pltpu.SemaphoreType.DMA),
            (pltpu.SemaphoreType.DMA, pltpu.SemaphoreType.DMA),
        ],
        compiler_params=pltpu.CompilerParams(
            use_tc_tiling_on_sc=False, needs_layout_passes=False),
    )


def _padrows(a):
    return jnp.pad(a, ((0, TP - T), (0, 0)))


def kernel(x):
    n = jnp.arange(WIN, dtype=jnp.float32)
    hann = 0.5 - 0.5 * jnp.cos(2.0 * jnp.pi * n / WIN)
    h2 = jnp.broadcast_to(hann[None, :], (8, NFFT))
    xp = jnp.pad(x, (NFFT // 2, NFFT // 2), mode="reflect")
    c = jnp.zeros((CROWS, HOP), jnp.float32).at[:xp.shape[0] // HOP].set(
        xp.reshape(-1, HOP))
    frames = _frames_stage(c, h2)
    spec = jnp.fft.rfft(frames[:T], n=NFFT, axis=-1)
    ph = jnp.angle(spec)
    inst = jnp.diff(ph, axis=0)
    instp = jnp.concatenate([inst, inst[-1:]], axis=0)
    vre, vim, kk = _reassign_stage(
        _padrows(spec.real), _padrows(spec.imag), _padrows(instp))
    outre, outim = _sc_scatter()(
        vre.reshape(TP * FP), vim.reshape(TP * FP), kk.reshape(TP * FP))
    z = lax.complex(outre.reshape(TP, FP), outim.reshape(TP, FP))
    return z.T[:FBINS, :T]
